# Initial kernel scaffold; baseline (speedup 1.0000x reference)
import jax
import jax.numpy as jnp
from jax.experimental import pallas as pl


def kernel(features, edge_index, kernel_ids, W1, gamma1, beta1, W2, gamma2, beta2, fc1_w, fc1_b, fc2_w, fc2_b):
    raise NotImplementedError("write your pallas kernel here")


# speedup vs baseline: 5.9547x; 5.9547x over previous
import functools

import jax
import jax.numpy as jnp
from jax import lax
from jax.experimental import pallas as pl
from jax.experimental.pallas import tpu as pltpu
from jax.experimental.pallas import tpu_sc as plsc

N = 10000
C = 128
E = 320000
K = 27

NC = 2
NS = 16
NW = NC * NS
EPW = E // NW
CHUNK = 80
NCHUNK = EPW // CHUNK
N_PAD = 10240
ROWS_PER_TILE = N_PAD // NS
LANES = 16

BN = 2000


def _mm_body(x_ref, w_ref, o_ref):
    o_ref[0] = jnp.dot(x_ref[...], w_ref[0],
                       preferred_element_type=jnp.float32)


def _mm(x, w):
    return pl.pallas_call(
        _mm_body,
        grid=(N // BN, K),
        in_specs=[
            pl.BlockSpec((BN, C), lambda n, k: (n, 0)),
            pl.BlockSpec((1, C, C), lambda n, k: (k, 0, 0)),
        ],
        out_specs=pl.BlockSpec((1, BN, C), lambda n, k: (k, n, 0)),
        out_shape=jax.ShapeDtypeStruct((K, N, C), jnp.float32),
    )(x, w)


def _bn_relu_body(p_ref, g_ref, b_ref, o_ref):
    x = p_ref[0] + p_ref[1]
    mu = jnp.mean(x, axis=0, keepdims=True)
    var = jnp.mean((x - mu) ** 2, axis=0, keepdims=True)
    xn = (x - mu) * lax.rsqrt(var + 1e-5) * g_ref[...] + b_ref[...]
    o_ref[...] = jnp.maximum(xn, 0.0)


def _bn_relu(partials, gamma, beta):
    return pl.pallas_call(
        _bn_relu_body,
        out_shape=jax.ShapeDtypeStruct((N, C), jnp.float32),
    )(partials, gamma.reshape(1, C), beta.reshape(1, C))


def _final_body(p_ref, id_ref, g_ref, b_ref, w1_ref, b1_ref, w2_ref, b2_ref,
                o_ref):
    x = p_ref[0] + p_ref[1]
    mu = jnp.mean(x, axis=0, keepdims=True)
    var = jnp.mean((x - mu) ** 2, axis=0, keepdims=True)
    xn = (x - mu) * lax.rsqrt(var + 1e-5) * g_ref[...] + b_ref[...]
    d = jnp.mean(xn, axis=0, keepdims=True)
    t = jnp.dot(d, w1_ref[...], preferred_element_type=jnp.float32)
    t = jnp.maximum(t + b1_ref[...], 0.0)
    u = jnp.dot(t, w2_ref[...], preferred_element_type=jnp.float32)
    s = 1.0 / (1.0 + jnp.exp(-(u + b2_ref[...])))
    o_ref[...] = jnp.maximum(xn * s + id_ref[...], 0.0)


def _final(partials, identity, gamma, beta, fc1_w, fc1_b, fc2_w, fc2_b):
    r = fc1_w.shape[1]
    return pl.pallas_call(
        _final_body,
        out_shape=jax.ShapeDtypeStruct((N, C), jnp.float32),
    )(partials, identity, gamma.reshape(1, C), beta.reshape(1, C),
      fc1_w, fc1_b.reshape(1, r), fc2_w, fc2_b.reshape(1, C))


SCH = 25
NSUPER = NCHUNK // SCH


def _sc_conv_body(g_hbm, src_hbm, dst_hbm, kid_hbm, out_hbm,
                  gidx_v, dst_v, rows0_v, rows1_v, acc, sem0, sem1):
    c = lax.axis_index("c")
    s = lax.axis_index("s")
    wid = s * NC + c

    def zero_body(i, carry):
        for j in range(C // LANES):
            rows0_v[i, pl.ds(j * LANES, LANES)] = jnp.zeros((LANES,),
                                                            jnp.float32)
        return carry

    lax.fori_loop(0, CHUNK, zero_body, 0)
    base = s * ROWS_PER_TILE
    for i in range(ROWS_PER_TILE // CHUNK):
        pltpu.sync_copy(rows0_v, acc.at[pl.ds(base + i * CHUNK, CHUNK)])
    plsc.subcore_barrier()

    def super_body(u, carry):
        pltpu.sync_copy(src_hbm.at[wid, u], gidx_v)
        pltpu.sync_copy(kid_hbm.at[wid, u], dst_v)

        def gidx_body(i, carry2):
            for j in range(CHUNK // LANES):
                sl = pl.ds(j * LANES, LANES)
                gidx_v[i, sl] = dst_v[i, sl] * N + gidx_v[i, sl]
            return carry2

        lax.fori_loop(0, SCH, gidx_body, 0)
        pltpu.sync_copy(dst_hbm.at[wid, u], dst_v)

        def chunk_body(i, carry2):
            pltpu.async_copy(g_hbm.at[gidx_v.at[i]], rows0_v, sem0).wait()
            pltpu.sync_copy(rows0_v, acc.at[dst_v.at[i]], add=True)
            return carry2

        lax.fori_loop(0, SCH, chunk_body, 0)
        return carry

    lax.fori_loop(0, NSUPER, super_body, 0)
    plsc.subcore_barrier()

    nout = jnp.minimum(ROWS_PER_TILE, N - base) // CHUNK

    def out_body(i, carry):
        off = base + i * CHUNK
        pltpu.sync_copy(acc.at[pl.ds(off, CHUNK)], rows0_v)
        pltpu.sync_copy(rows0_v, out_hbm.at[c, pl.ds(off, CHUNK)])
        return carry

    lax.fori_loop(0, nout, out_body, 0)


@functools.cache
def _make_sc_conv():
    return pl.kernel(
        _sc_conv_body,
        out_type=jax.ShapeDtypeStruct((NC, N, C), jnp.float32),
        mesh=plsc.VectorSubcoreMesh(core_axis_name="c", subcore_axis_name="s"),
        scratch_types=[
            pltpu.VMEM((SCH, CHUNK), jnp.int32),
            pltpu.VMEM((SCH, CHUNK), jnp.int32),
            pltpu.VMEM((CHUNK, C), jnp.float32),
            pltpu.VMEM((CHUNK, C), jnp.float32),
            pltpu.VMEM_SHARED((N_PAD, C), jnp.float32),
            pltpu.SemaphoreType.DMA,
            pltpu.SemaphoreType.DMA,
        ],
    )


def _sc_conv(g_flat, src, dst, kid):
    return _make_sc_conv()(g_flat, src, dst, kid)


def kernel(features, edge_index, kernel_ids, W1, gamma1, beta1,
           W2, gamma2, beta2, fc1_w, fc1_b, fc2_w, fc2_b):
    src = edge_index[0].reshape(NW, NSUPER, SCH, CHUNK)
    dst = edge_index[1].reshape(NW, NSUPER, SCH, CHUNK)
    kid = kernel_ids.reshape(NW, NSUPER, SCH, CHUNK)

    g1 = _mm(features, W1).reshape(K * N, C)
    p1 = _sc_conv(g1, src, dst, kid)
    x1 = _bn_relu(p1, gamma1, beta1)

    g2 = _mm(x1, W2).reshape(K * N, C)
    p2 = _sc_conv(g2, src, dst, kid)
    return _final(p2, features, gamma2, beta2, fc1_w, fc1_b, fc2_w, fc2_b)


# speedup vs baseline: 7.7963x; 1.3093x over previous
import functools

import jax
import jax.numpy as jnp
from jax import lax
from jax.experimental import pallas as pl
from jax.experimental.pallas import tpu as pltpu
from jax.experimental.pallas import tpu_sc as plsc

N = 10000
C = 128
E = 320000
K = 27

NC = 2
NS = 16
NW = NC * NS
EPW = E // NW
CHUNK = 80
NCHUNK = EPW // CHUNK
N_PAD = 10240
ROWS_PER_TILE = N_PAD // NS
LANES = 16

BN = 2000


def _mm_body(x_ref, w_ref, o_ref):
    o_ref[0] = jnp.dot(x_ref[...], w_ref[0],
                       preferred_element_type=jnp.float32)


def _mm(x, w):
    return pl.pallas_call(
        _mm_body,
        grid=(N // BN, K),
        in_specs=[
            pl.BlockSpec((BN, C), lambda n, k: (n, 0)),
            pl.BlockSpec((1, C, C), lambda n, k: (k, 0, 0)),
        ],
        out_specs=pl.BlockSpec((1, BN, C), lambda n, k: (k, n, 0)),
        out_shape=jax.ShapeDtypeStruct((K, N, C), jnp.float32),
    )(x, w)


def _bn_relu_body(p_ref, g_ref, b_ref, o_ref):
    x = p_ref[0] + p_ref[1]
    mu = jnp.mean(x, axis=0, keepdims=True)
    var = jnp.mean((x - mu) ** 2, axis=0, keepdims=True)
    xn = (x - mu) * lax.rsqrt(var + 1e-5) * g_ref[...] + b_ref[...]
    o_ref[...] = jnp.maximum(xn, 0.0)


def _bn_relu(partials, gamma, beta):
    return pl.pallas_call(
        _bn_relu_body,
        out_shape=jax.ShapeDtypeStruct((N, C), jnp.float32),
    )(partials, gamma.reshape(1, C), beta.reshape(1, C))


def _final_body(p_ref, id_ref, g_ref, b_ref, w1_ref, b1_ref, w2_ref, b2_ref,
                o_ref):
    x = p_ref[0] + p_ref[1]
    mu = jnp.mean(x, axis=0, keepdims=True)
    var = jnp.mean((x - mu) ** 2, axis=0, keepdims=True)
    xn = (x - mu) * lax.rsqrt(var + 1e-5) * g_ref[...] + b_ref[...]
    d = jnp.mean(xn, axis=0, keepdims=True)
    t = jnp.dot(d, w1_ref[...], preferred_element_type=jnp.float32)
    t = jnp.maximum(t + b1_ref[...], 0.0)
    u = jnp.dot(t, w2_ref[...], preferred_element_type=jnp.float32)
    s = 1.0 / (1.0 + jnp.exp(-(u + b2_ref[...])))
    o_ref[...] = jnp.maximum(xn * s + id_ref[...], 0.0)


def _final(partials, identity, gamma, beta, fc1_w, fc1_b, fc2_w, fc2_b):
    r = fc1_w.shape[1]
    return pl.pallas_call(
        _final_body,
        out_shape=jax.ShapeDtypeStruct((N, C), jnp.float32),
    )(partials, identity, gamma.reshape(1, C), beta.reshape(1, C),
      fc1_w, fc1_b.reshape(1, r), fc2_w, fc2_b.reshape(1, C))


SCH = 25
NSUPER = NCHUNK // SCH


def _sc_conv_body(g_hbm, src_hbm, dst_hbm, kid_hbm, out_hbm,
                  gidx_v, dst_v, rows0_v, rows1_v, acc, sem0, sem1):
    c = lax.axis_index("c")
    s = lax.axis_index("s")
    wid = s * NC + c

    def zero_body(i, carry):
        for j in range(C // LANES):
            rows0_v[i, pl.ds(j * LANES, LANES)] = jnp.zeros((LANES,),
                                                            jnp.float32)
        return carry

    lax.fori_loop(0, CHUNK, zero_body, 0)
    base = s * ROWS_PER_TILE
    for i in range(ROWS_PER_TILE // CHUNK):
        pltpu.sync_copy(rows0_v, acc.at[pl.ds(base + i * CHUNK, CHUNK)])
    plsc.subcore_barrier()

    def super_body(u, carry):
        pltpu.sync_copy(src_hbm.at[wid, u], gidx_v)
        pltpu.sync_copy(kid_hbm.at[wid, u], dst_v)

        def gidx_body(i, carry2):
            for j in range(CHUNK // LANES):
                sl = pl.ds(j * LANES, LANES)
                gidx_v[i, sl] = dst_v[i, sl] * N + gidx_v[i, sl]
            return carry2

        lax.fori_loop(0, SCH, gidx_body, 0)
        pltpu.sync_copy(dst_hbm.at[wid, u], dst_v)

        def start(i, rows, sem):
            pltpu.async_copy(g_hbm.at[gidx_v.at[i]], rows, sem)

        def finish(i, rows, sem):
            pltpu.make_async_copy(g_hbm.at[gidx_v.at[i]], rows, sem).wait()
            pltpu.sync_copy(rows, acc.at[dst_v.at[i]], add=True)

        start(0, rows0_v, sem0)

        def pipe_body(t, carry2):
            i = 2 * t
            start(i + 1, rows1_v, sem1)
            finish(i, rows0_v, sem0)
            start(i + 2, rows0_v, sem0)
            finish(i + 1, rows1_v, sem1)
            return carry2

        lax.fori_loop(0, (SCH - 1) // 2, pipe_body, 0)
        finish(SCH - 1, rows0_v, sem0)
        return carry

    lax.fori_loop(0, NSUPER, super_body, 0)
    plsc.subcore_barrier()

    nout = jnp.minimum(ROWS_PER_TILE, N - base) // CHUNK

    def out_body(i, carry):
        off = base + i * CHUNK
        pltpu.sync_copy(acc.at[pl.ds(off, CHUNK)], rows0_v)
        pltpu.sync_copy(rows0_v, out_hbm.at[c, pl.ds(off, CHUNK)])
        return carry

    lax.fori_loop(0, nout, out_body, 0)


@functools.cache
def _make_sc_conv():
    return pl.kernel(
        _sc_conv_body,
        out_type=jax.ShapeDtypeStruct((NC, N, C), jnp.float32),
        mesh=plsc.VectorSubcoreMesh(core_axis_name="c", subcore_axis_name="s"),
        scratch_types=[
            pltpu.VMEM((SCH, CHUNK), jnp.int32),
            pltpu.VMEM((SCH, CHUNK), jnp.int32),
            pltpu.VMEM((CHUNK, C), jnp.float32),
            pltpu.VMEM((CHUNK, C), jnp.float32),
            pltpu.VMEM_SHARED((N_PAD, C), jnp.float32),
            pltpu.SemaphoreType.DMA,
            pltpu.SemaphoreType.DMA,
        ],
    )


def _sc_conv(g_flat, src, dst, kid):
    return _make_sc_conv()(g_flat, src, dst, kid)


def kernel(features, edge_index, kernel_ids, W1, gamma1, beta1,
           W2, gamma2, beta2, fc1_w, fc1_b, fc2_w, fc2_b):
    src = edge_index[0].reshape(NW, NSUPER, SCH, CHUNK)
    dst = edge_index[1].reshape(NW, NSUPER, SCH, CHUNK)
    kid = kernel_ids.reshape(NW, NSUPER, SCH, CHUNK)

    g1 = _mm(features, W1).reshape(K * N, C)
    p1 = _sc_conv(g1, src, dst, kid)
    x1 = _bn_relu(p1, gamma1, beta1)

    g2 = _mm(x1, W2).reshape(K * N, C)
    p2 = _sc_conv(g2, src, dst, kid)
    return _final(p2, features, gamma2, beta2, fc1_w, fc1_b, fc2_w, fc2_b)


# speedup vs baseline: 8.2982x; 1.0644x over previous
import functools

import jax
import jax.numpy as jnp
from jax import lax
from jax.experimental import pallas as pl
from jax.experimental.pallas import tpu as pltpu
from jax.experimental.pallas import tpu_sc as plsc

N = 10000
C = 128
E = 320000
K = 27

NC = 2
NS = 16
NW = NC * NS
EPW = E // NW
CHUNK = 40
NCHUNK = EPW // CHUNK
N_PAD = 10240
ROWS_PER_TILE = N_PAD // NS
LANES = 16

BN = 2000


def _mm_body(x_ref, w_ref, o_ref):
    o_ref[0] = jnp.dot(x_ref[...], w_ref[0],
                       preferred_element_type=jnp.float32)


def _mm(x, w):
    return pl.pallas_call(
        _mm_body,
        grid=(N // BN, K),
        in_specs=[
            pl.BlockSpec((BN, C), lambda n, k: (n, 0)),
            pl.BlockSpec((1, C, C), lambda n, k: (k, 0, 0)),
        ],
        out_specs=pl.BlockSpec((1, BN, C), lambda n, k: (k, n, 0)),
        out_shape=jax.ShapeDtypeStruct((K, N, C), jnp.float32),
    )(x, w)


def _bn_relu_body(p_ref, g_ref, b_ref, o_ref):
    x = p_ref[0] + p_ref[1]
    mu = jnp.mean(x, axis=0, keepdims=True)
    var = jnp.mean((x - mu) ** 2, axis=0, keepdims=True)
    xn = (x - mu) * lax.rsqrt(var + 1e-5) * g_ref[...] + b_ref[...]
    o_ref[...] = jnp.maximum(xn, 0.0)


def _bn_relu(partials, gamma, beta):
    return pl.pallas_call(
        _bn_relu_body,
        out_shape=jax.ShapeDtypeStruct((N, C), jnp.float32),
    )(partials, gamma.reshape(1, C), beta.reshape(1, C))


def _final_body(p_ref, id_ref, g_ref, b_ref, w1_ref, b1_ref, w2_ref, b2_ref,
                o_ref):
    x = p_ref[0] + p_ref[1]
    mu = jnp.mean(x, axis=0, keepdims=True)
    var = jnp.mean((x - mu) ** 2, axis=0, keepdims=True)
    xn = (x - mu) * lax.rsqrt(var + 1e-5) * g_ref[...] + b_ref[...]
    d = jnp.mean(xn, axis=0, keepdims=True)
    t = jnp.dot(d, w1_ref[...], preferred_element_type=jnp.float32)
    t = jnp.maximum(t + b1_ref[...], 0.0)
    u = jnp.dot(t, w2_ref[...], preferred_element_type=jnp.float32)
    s = 1.0 / (1.0 + jnp.exp(-(u + b2_ref[...])))
    o_ref[...] = jnp.maximum(xn * s + id_ref[...], 0.0)


def _final(partials, identity, gamma, beta, fc1_w, fc1_b, fc2_w, fc2_b):
    r = fc1_w.shape[1]
    return pl.pallas_call(
        _final_body,
        out_shape=jax.ShapeDtypeStruct((N, C), jnp.float32),
    )(partials, identity, gamma.reshape(1, C), beta.reshape(1, C),
      fc1_w, fc1_b.reshape(1, r), fc2_w, fc2_b.reshape(1, C))


SCH = 50
NSUPER = NCHUNK // SCH
DEPTH = 4


def _sc_conv_body(g_hbm, gidx_hbm, dst_hbm, out_hbm,
                  gidx_v, dst_v, rows0_v, rows1_v, rows2_v, rows3_v, acc,
                  sem0, sem1, sem2, sem3):
    c = lax.axis_index("c")
    s = lax.axis_index("s")
    wid = s * NC + c
    slots = ((rows0_v, sem0), (rows1_v, sem1), (rows2_v, sem2),
             (rows3_v, sem3))

    def zero_body(i, carry):
        for j in range(C // LANES):
            rows0_v[i, pl.ds(j * LANES, LANES)] = jnp.zeros((LANES,),
                                                            jnp.float32)
        return carry

    lax.fori_loop(0, CHUNK, zero_body, 0)
    base = s * ROWS_PER_TILE
    for i in range(ROWS_PER_TILE // CHUNK):
        pltpu.sync_copy(rows0_v, acc.at[pl.ds(base + i * CHUNK, CHUNK)])
    plsc.subcore_barrier()

    def super_body(u, carry):
        pltpu.sync_copy(gidx_hbm.at[wid, u], gidx_v)
        pltpu.sync_copy(dst_hbm.at[wid, u], dst_v)

        def start(i, q):
            pltpu.async_copy(g_hbm.at[gidx_v.at[i]], slots[q][0],
                             slots[q][1])

        def finish(i, q):
            r, sm = slots[q]
            pltpu.make_async_copy(g_hbm.at[gidx_v.at[i]], r, sm).wait()
            pltpu.sync_copy(r, acc.at[dst_v.at[i]], add=True)

        for q in range(DEPTH):
            start(q, q)

        def pipe_body(t, carry2):
            for q in range(DEPTH):
                i = DEPTH * t + q
                finish(i, q)
                start(i + DEPTH, q)
            return carry2

        nmain = (SCH - DEPTH - 2) // DEPTH
        lax.fori_loop(0, nmain, pipe_body, 0)
        for i in range(nmain * DEPTH, SCH):
            q = i % DEPTH
            finish(i, q)
            if i + DEPTH < SCH:
                start(i + DEPTH, q)
        return carry

    lax.fori_loop(0, NSUPER, super_body, 0)
    plsc.subcore_barrier()

    nout = jnp.minimum(ROWS_PER_TILE, N - base) // CHUNK

    def out_body(i, carry):
        off = base + i * CHUNK
        pltpu.sync_copy(acc.at[pl.ds(off, CHUNK)], rows0_v)
        pltpu.sync_copy(rows0_v, out_hbm.at[c, pl.ds(off, CHUNK)])
        return carry

    lax.fori_loop(0, nout, out_body, 0)


@functools.cache
def _make_sc_conv():
    return pl.kernel(
        _sc_conv_body,
        out_type=jax.ShapeDtypeStruct((NC, N, C), jnp.float32),
        mesh=plsc.VectorSubcoreMesh(core_axis_name="c", subcore_axis_name="s"),
        scratch_types=[
            pltpu.VMEM((SCH, CHUNK), jnp.int32),
            pltpu.VMEM((SCH, CHUNK), jnp.int32),
            pltpu.VMEM((CHUNK, C), jnp.float32),
            pltpu.VMEM((CHUNK, C), jnp.float32),
            pltpu.VMEM((CHUNK, C), jnp.float32),
            pltpu.VMEM((CHUNK, C), jnp.float32),
            pltpu.VMEM_SHARED((N_PAD, C), jnp.float32),
            pltpu.SemaphoreType.DMA,
            pltpu.SemaphoreType.DMA,
            pltpu.SemaphoreType.DMA,
            pltpu.SemaphoreType.DMA,
        ],
    )


def _sc_conv(g_flat, gidx, dst):
    return _make_sc_conv()(g_flat, gidx, dst)


def _gidx_body(s_ref, k_ref, o_ref):
    o_ref[...] = k_ref[...] * N + s_ref[...]


def _gidx(src, kid):
    return pl.pallas_call(
        _gidx_body,
        out_shape=jax.ShapeDtypeStruct((E // C, C), jnp.int32),
    )(src.reshape(E // C, C), kid.reshape(E // C, C))


def kernel(features, edge_index, kernel_ids, W1, gamma1, beta1,
           W2, gamma2, beta2, fc1_w, fc1_b, fc2_w, fc2_b):
    gidx = _gidx(edge_index[0], kernel_ids).reshape(NW, NSUPER, SCH, CHUNK)
    dst = edge_index[1].reshape(NW, NSUPER, SCH, CHUNK)

    g1 = _mm(features, W1).reshape(K * N, C)
    p1 = _sc_conv(g1, gidx, dst)
    x1 = _bn_relu(p1, gamma1, beta1)

    g2 = _mm(x1, W2).reshape(K * N, C)
    p2 = _sc_conv(g2, gidx, dst)
    return _final(p2, features, gamma2, beta2, fc1_w, fc1_b, fc2_w, fc2_b)
